# baseline (speedup 1.0000x reference)
import jax
import jax.numpy as jnp
import numpy as np
from jax.experimental import pallas as pl
from jax.experimental.pallas import tpu as pltpu

_SIGMA = 0.2
_B = 64
_H = 28
_W = 28
_C = 384


def _consts():
    cpu = jax.devices("cpu")[0]
    with jax.default_device(cpu):
        key = jax.random.key(42)
        k1, k2, k3 = jax.random.split(key, 3)
        indices = jax.random.permutation(k1, _B)
        rs = (1, _H, _W, _C)
        r = jax.random.normal(k2, rs, dtype=jnp.float16) * jnp.float16(_SIGMA)
        theta = jax.random.uniform(
            k3, rs, dtype=jnp.float16, minval=-np.pi, maxval=np.pi)
        a = (jnp.float16(1.0) + r * jnp.cos(theta)).astype(jnp.float32)
        b = (r * jnp.sin(theta)).astype(jnp.float32)
        a_np = np.asarray(a).reshape(_H, _W, _C)
        b_np = np.asarray(b).reshape(_H, _W, _C)
        perm_np = np.asarray(indices, dtype=np.int32)
    return a_np, b_np, perm_np


_A_NP, _B_NP, _PERM_NP = _consts()

_HH = _H // 2


def _mix_body(perm_ref, xs0, xs1, xp0, xp1, a_ref, b_ref, out_ref):
    del perm_ref
    out_ref[0, :_HH] = xs0[0] * a_ref[:_HH] + xp0[0] * b_ref[:_HH]
    out_ref[0, _HH:] = xs1[0] * a_ref[_HH:] + xp1[0] * b_ref[_HH:]


def kernel(x):
    a = jnp.asarray(_A_NP)
    b = jnp.asarray(_B_NP)
    perm = jnp.asarray(_PERM_NP)

    half = (1, _HH, _W, _C)
    grid_spec = pltpu.PrefetchScalarGridSpec(
        num_scalar_prefetch=1,
        grid=(_B,),
        in_specs=[
            pl.BlockSpec(half, lambda i, p: (i, 0, 0, 0)),
            pl.BlockSpec(half, lambda i, p: (i, 1, 0, 0)),
            pl.BlockSpec(half, lambda i, p: (p[i], 0, 0, 0)),
            pl.BlockSpec(half, lambda i, p: (p[i], 1, 0, 0)),
            pl.BlockSpec((_H, _W, _C), lambda i, p: (0, 0, 0)),
            pl.BlockSpec((_H, _W, _C), lambda i, p: (0, 0, 0)),
        ],
        out_specs=pl.BlockSpec((1, _H, _W, _C), lambda i, p: (i, 0, 0, 0)),
    )
    y = pl.pallas_call(
        _mix_body,
        grid_spec=grid_spec,
        out_shape=jax.ShapeDtypeStruct((_B, _H, _W, _C), jnp.float32),
    )(perm, x, x, x, x, a, b)
    return y


# speedup vs baseline: 1.0330x; 1.0330x over previous
import jax
import jax.numpy as jnp
import numpy as np
from jax.experimental import pallas as pl
from jax.experimental.pallas import tpu as pltpu

_SIGMA = 0.2
_B = 64
_H = 28
_W = 28
_C = 384


def _consts():
    cpu = jax.devices("cpu")[0]
    with jax.default_device(cpu):
        key = jax.random.key(42)
        k1, k2, k3 = jax.random.split(key, 3)
        indices = jax.random.permutation(k1, _B)
        rs = (1, _H, _W, _C)
        r = jax.random.normal(k2, rs, dtype=jnp.float16) * jnp.float16(_SIGMA)
        theta = jax.random.uniform(
            k3, rs, dtype=jnp.float16, minval=-np.pi, maxval=np.pi)
        a = (jnp.float16(1.0) + r * jnp.cos(theta)).astype(jnp.float32)
        b = (r * jnp.sin(theta)).astype(jnp.float32)
        a_np = np.asarray(a).reshape(_H, _W, _C)
        b_np = np.asarray(b).reshape(_H, _W, _C)
        perm_np = np.asarray(indices, dtype=np.int32)
    return a_np, b_np, perm_np


_A_NP, _B_NP, _PERM_NP = _consts()


def _mix_body(perm_ref, xs, xp0, xp1, a_ref, b_ref, out_ref):
    del perm_ref
    out_ref[0] = xs[0] * a_ref[...] + xp0[0] * b_ref[...]
    out_ref[1] = xs[1] * a_ref[...] + xp1[0] * b_ref[...]


def kernel(x):
    a = jnp.asarray(_A_NP)
    b = jnp.asarray(_B_NP)
    perm = jnp.asarray(_PERM_NP)

    one = (1, _H, _W, _C)
    grid_spec = pltpu.PrefetchScalarGridSpec(
        num_scalar_prefetch=1,
        grid=(_B // 2,),
        in_specs=[
            pl.BlockSpec((2, _H, _W, _C), lambda i, p: (i, 0, 0, 0)),
            pl.BlockSpec(one, lambda i, p: (p[2 * i], 0, 0, 0)),
            pl.BlockSpec(one, lambda i, p: (p[2 * i + 1], 0, 0, 0)),
            pl.BlockSpec((_H, _W, _C), lambda i, p: (0, 0, 0)),
            pl.BlockSpec((_H, _W, _C), lambda i, p: (0, 0, 0)),
        ],
        out_specs=pl.BlockSpec((2, _H, _W, _C), lambda i, p: (i, 0, 0, 0)),
    )
    y = pl.pallas_call(
        _mix_body,
        grid_spec=grid_spec,
        out_shape=jax.ShapeDtypeStruct((_B, _H, _W, _C), jnp.float32),
    )(perm, x, x, x, a, b)
    return y


# speedup vs baseline: 1.1599x; 1.1228x over previous
import jax
import jax.numpy as jnp
import numpy as np
from jax import lax
from jax.experimental import pallas as pl
from jax.experimental.pallas import tpu as pltpu

_SIGMA = 0.2
_B = 64
_H = 28
_W = 28
_C = 384
_NBUF = 12
_SLAB = 7


def _consts():
    cpu = jax.devices("cpu")[0]
    with jax.default_device(cpu):
        key = jax.random.key(42)
        k1, k2, k3 = jax.random.split(key, 3)
        indices = jax.random.permutation(k1, _B)
        rs = (1, _H, _W, _C)
        r = jax.random.normal(k2, rs, dtype=jnp.float16) * jnp.float16(_SIGMA)
        theta = jax.random.uniform(
            k3, rs, dtype=jnp.float16, minval=-np.pi, maxval=np.pi)
        a = (jnp.float16(1.0) + r * jnp.cos(theta)).astype(jnp.float32)
        b = (r * jnp.sin(theta)).astype(jnp.float32)
        a_np = np.asarray(a).reshape(_H, _W, _C)
        b_np = np.asarray(b).reshape(_H, _W, _C)
        perm_np = np.asarray(indices, dtype=np.int32)
    return a_np, b_np, perm_np


_A_NP, _B_NP, _PERM_NP = _consts()


def _schedule():
    perm = [int(v) for v in _PERM_NP]
    seen = [False] * _B
    fetches, steps = [], []
    for i in range(_B):
        if seen[i]:
            continue
        cyc = []
        j = i
        while not seen[j]:
            seen[j] = True
            cyc.append(j)
            j = perm[j]
        base = len(fetches)
        fetches.extend(cyc)
        if len(cyc) == 1:
            steps.append((cyc[0], base, base))
        else:
            fetches.append(cyc[0])
            for j2 in range(len(cyc)):
                steps.append((cyc[j2], base + j2, base + j2 + 1))
    first_use = {}
    last_use = {}
    for t, (_, sq, pq) in enumerate(steps):
        for q in (sq, pq):
            first_use.setdefault(q, t)
            last_use[q] = t
    start_at = {}
    started = 0
    for t in range(len(steps)):
        while started < len(fetches) and (
                started < _NBUF or last_use[started - _NBUF] <= t - 1):
            start_at[started] = t
            started += 1
        _, sq, pq = steps[t]
        for q in (sq, pq):
            assert start_at[q] <= t, (q, t)
    assert started == len(fetches)
    return fetches, steps, first_use, last_use, start_at


_FETCHES, _STEPS, _FIRST_USE, _LAST_USE, _START_AT = _schedule()


def _row_mix(dst, xs, xp, av, bv):
    def h_body(h, c):
        sl = pl.ds(h * _SLAB, _SLAB)
        dst[sl] = xs[sl] * av[sl] + xp[sl] * bv[sl]
        return c
    lax.fori_loop(0, _H // _SLAB, h_body, 0, unroll=False)


_NST = 4


def _mix_body(x_hbm0, x_hbm1, a_v, b_v, y_hbm, *scratch):
    bufs = scratch[:_NBUF]
    st = scratch[_NBUF:_NBUF + _NST]
    sems = scratch[_NBUF + _NST:_NBUF + _NST + _NBUF]
    so = scratch[_NBUF + _NST + _NBUF:]
    xh = (x_hbm0, x_hbm1)

    starts_by_step = [[] for _ in range(len(_STEPS))]
    for q, t0 in _START_AT.items():
        starts_by_step[t0].append(q)

    def start_fetch(q):
        s = q % _NBUF
        pltpu.make_async_copy(
            xh[q % 2].at[_FETCHES[q]], bufs[s], sems[s]).start()

    def wait_fetch(q):
        s = q % _NBUF
        pltpu.make_async_copy(
            xh[q % 2].at[_FETCHES[q]], bufs[s], sems[s]).wait()

    for t, (orow, sq, pq) in enumerate(_STEPS):
        for q in starts_by_step[t]:
            start_fetch(q)
        for q in {sq, pq}:
            if _FIRST_USE[q] == t:
                wait_fetch(q)
        if t >= _NST:
            prow = _STEPS[t - _NST][0]
            pltpu.make_async_copy(
                st[t % _NST], y_hbm.at[prow], so[t % _NST]).wait()
        _row_mix(st[t % _NST], bufs[sq % _NBUF], bufs[pq % _NBUF], a_v, b_v)
        pltpu.make_async_copy(
            st[t % _NST], y_hbm.at[orow], so[t % _NST]).start()

    n = len(_STEPS)
    for t in range(n - _NST, n):
        pltpu.make_async_copy(
            st[t % _NST], y_hbm.at[_STEPS[t][0]], so[t % _NST]).wait()


def kernel(x):
    a = jnp.asarray(_A_NP)
    b = jnp.asarray(_B_NP)
    scratch = (
        [pltpu.VMEM((_H, _W, _C), jnp.float32)] * (_NBUF + _NST)
        + [pltpu.SemaphoreType.DMA] * (_NBUF + _NST)
    )
    y = pl.pallas_call(
        _mix_body,
        grid=(1,),
        in_specs=[
            pl.BlockSpec(memory_space=pl.ANY),
            pl.BlockSpec(memory_space=pl.ANY),
            pl.BlockSpec((_H, _W, _C), lambda i: (0, 0, 0)),
            pl.BlockSpec((_H, _W, _C), lambda i: (0, 0, 0)),
        ],
        out_specs=pl.BlockSpec(memory_space=pl.ANY),
        out_shape=jax.ShapeDtypeStruct((_B, _H, _W, _C), jnp.float32),
        scratch_shapes=scratch,
    )(x, x, a, b)
    return y


# speedup vs baseline: 1.1621x; 1.0019x over previous
import jax
import jax.numpy as jnp
import numpy as np
from jax import lax
from jax.experimental import pallas as pl
from jax.experimental.pallas import tpu as pltpu

_SIGMA = 0.2
_B = 64
_H = 28
_W = 28
_C = 384
_NBUF = 12
_SLAB = 7


def _consts():
    cpu = jax.devices("cpu")[0]
    with jax.default_device(cpu):
        key = jax.random.key(42)
        k1, k2, k3 = jax.random.split(key, 3)
        indices = jax.random.permutation(k1, _B)
        rs = (1, _H, _W, _C)
        r = jax.random.normal(k2, rs, dtype=jnp.float16) * jnp.float16(_SIGMA)
        theta = jax.random.uniform(
            k3, rs, dtype=jnp.float16, minval=-np.pi, maxval=np.pi)
        a = (jnp.float16(1.0) + r * jnp.cos(theta)).astype(jnp.float32)
        b = (r * jnp.sin(theta)).astype(jnp.float32)
        a_np = np.asarray(a.astype(jnp.bfloat16)).reshape(_H, _W, _C)
        b_np = np.asarray(b.astype(jnp.bfloat16)).reshape(_H, _W, _C)
        perm_np = np.asarray(indices, dtype=np.int32)
    return a_np, b_np, perm_np


_A_NP, _B_NP, _PERM_NP = _consts()


def _schedule():
    perm = [int(v) for v in _PERM_NP]
    seen = [False] * _B
    fetches, steps = [], []
    for i in range(_B):
        if seen[i]:
            continue
        cyc = []
        j = i
        while not seen[j]:
            seen[j] = True
            cyc.append(j)
            j = perm[j]
        base = len(fetches)
        fetches.extend(cyc)
        if len(cyc) == 1:
            steps.append((cyc[0], base, base))
        else:
            fetches.append(cyc[0])
            for j2 in range(len(cyc)):
                steps.append((cyc[j2], base + j2, base + j2 + 1))
    first_use = {}
    last_use = {}
    for t, (_, sq, pq) in enumerate(steps):
        for q in (sq, pq):
            first_use.setdefault(q, t)
            last_use[q] = t
    start_at = {}
    started = 0
    for t in range(len(steps)):
        while started < len(fetches) and (
                started < _NBUF or last_use[started - _NBUF] <= t - 1):
            start_at[started] = t
            started += 1
        _, sq, pq = steps[t]
        for q in (sq, pq):
            assert start_at[q] <= t, (q, t)
    assert started == len(fetches)
    return fetches, steps, first_use, last_use, start_at


_FETCHES, _STEPS, _FIRST_USE, _LAST_USE, _START_AT = _schedule()


def _row_mix(dst, xs, xp, av, bv):
    def h_body(h, c):
        sl = pl.ds(h * _SLAB, _SLAB)
        dst[sl] = (xs[sl] * av[sl].astype(jnp.float32)
                   + xp[sl] * bv[sl].astype(jnp.float32))
        return c
    lax.fori_loop(0, _H // _SLAB, h_body, 0, unroll=False)


_NST = 4


def _mix_body(x_hbm0, x_hbm1, a_v, b_v, y_hbm, *scratch):
    bufs = scratch[:_NBUF]
    st = scratch[_NBUF:_NBUF + _NST]
    sems = scratch[_NBUF + _NST:_NBUF + _NST + _NBUF]
    so = scratch[_NBUF + _NST + _NBUF:]
    xh = (x_hbm0, x_hbm1)

    starts_by_step = [[] for _ in range(len(_STEPS))]
    for q, t0 in _START_AT.items():
        starts_by_step[t0].append(q)

    def start_fetch(q):
        s = q % _NBUF
        pltpu.make_async_copy(
            xh[q % 2].at[_FETCHES[q]], bufs[s], sems[s]).start()

    def wait_fetch(q):
        s = q % _NBUF
        pltpu.make_async_copy(
            xh[q % 2].at[_FETCHES[q]], bufs[s], sems[s]).wait()

    for t, (orow, sq, pq) in enumerate(_STEPS):
        for q in starts_by_step[t]:
            start_fetch(q)
        for q in {sq, pq}:
            if _FIRST_USE[q] == t:
                wait_fetch(q)
        if t >= _NST:
            prow = _STEPS[t - _NST][0]
            pltpu.make_async_copy(
                st[t % _NST], y_hbm.at[prow], so[t % _NST]).wait()
        _row_mix(st[t % _NST], bufs[sq % _NBUF], bufs[pq % _NBUF], a_v, b_v)
        pltpu.make_async_copy(
            st[t % _NST], y_hbm.at[orow], so[t % _NST]).start()

    n = len(_STEPS)
    for t in range(n - _NST, n):
        pltpu.make_async_copy(
            st[t % _NST], y_hbm.at[_STEPS[t][0]], so[t % _NST]).wait()


def kernel(x):
    a = jnp.asarray(_A_NP)
    b = jnp.asarray(_B_NP)
    scratch = (
        [pltpu.VMEM((_H, _W, _C), jnp.float32)] * (_NBUF + _NST)
        + [pltpu.SemaphoreType.DMA] * (_NBUF + _NST)
    )
    y = pl.pallas_call(
        _mix_body,
        grid=(1,),
        in_specs=[
            pl.BlockSpec(memory_space=pl.ANY),
            pl.BlockSpec(memory_space=pl.ANY),
            pl.BlockSpec((_H, _W, _C), lambda i: (0, 0, 0)),
            pl.BlockSpec((_H, _W, _C), lambda i: (0, 0, 0)),
        ],
        out_specs=pl.BlockSpec(memory_space=pl.ANY),
        out_shape=jax.ShapeDtypeStruct((_B, _H, _W, _C), jnp.float32),
        scratch_shapes=scratch,
    )(x, x, a, b)
    return y


# speedup vs baseline: 1.1649x; 1.0024x over previous
import jax
import jax.numpy as jnp
import numpy as np
from jax import lax
from jax.experimental import pallas as pl
from jax.experimental.pallas import tpu as pltpu

_SIGMA = 0.2
_B = 64
_H = 28
_W = 28
_C = 384
_NBUF = 12
_SLAB = 7


def _consts():
    cpu = jax.devices("cpu")[0]
    with jax.default_device(cpu):
        key = jax.random.key(42)
        k1, k2, k3 = jax.random.split(key, 3)
        indices = jax.random.permutation(k1, _B)
        rs = (1, _H, _W, _C)
        r = jax.random.normal(k2, rs, dtype=jnp.float16) * jnp.float16(_SIGMA)
        theta = jax.random.uniform(
            k3, rs, dtype=jnp.float16, minval=-np.pi, maxval=np.pi)
        a = (jnp.float16(1.0) + r * jnp.cos(theta)).astype(jnp.float32)
        b = (r * jnp.sin(theta)).astype(jnp.float32)
        a_np = np.asarray(a.astype(jnp.bfloat16)).reshape(_H, _W, _C)
        b_np = np.asarray(b.astype(jnp.bfloat16)).reshape(_H, _W, _C)
        perm_np = np.asarray(indices, dtype=np.int32)
    return a_np, b_np, perm_np


_A_NP, _B_NP, _PERM_NP = _consts()


def _schedule():
    perm = [int(v) for v in _PERM_NP]
    seen = [False] * _B
    fetches, steps = [], []
    for i in range(_B):
        if seen[i]:
            continue
        cyc = []
        j = i
        while not seen[j]:
            seen[j] = True
            cyc.append(j)
            j = perm[j]
        base = len(fetches)
        fetches.extend(cyc)
        if len(cyc) == 1:
            steps.append((cyc[0], base, base))
        else:
            fetches.append(cyc[0])
            for j2 in range(len(cyc)):
                steps.append((cyc[j2], base + j2, base + j2 + 1))
    first_use = {}
    last_use = {}
    for t, (_, sq, pq) in enumerate(steps):
        for q in (sq, pq):
            first_use.setdefault(q, t)
            last_use[q] = t
    start_at = {}
    started = 0
    for t in range(len(steps)):
        while started < len(fetches) and (
                started < _NBUF or last_use[started - _NBUF] <= t - 1):
            start_at[started] = t
            started += 1
        _, sq, pq = steps[t]
        for q in (sq, pq):
            assert start_at[q] <= t, (q, t)
    assert started == len(fetches)
    return fetches, steps, first_use, last_use, start_at


_FETCHES, _STEPS, _FIRST_USE, _LAST_USE, _START_AT = _schedule()


def _row_mix(dst, xs, xp, av, bv):
    def h_body(h, c):
        sl = pl.ds(h * _SLAB, _SLAB)
        dst[sl] = (xs[sl] * av[sl].astype(jnp.float32)
                   + xp[sl] * bv[sl].astype(jnp.float32))
        return c
    lax.fori_loop(0, _H // _SLAB, h_body, 0, unroll=False)


_NST = 4


def _mix_body(x_hbm0, x_hbm1, a_v, b_v, y_hbm, *scratch):
    bufs = scratch[:_NBUF]
    st = scratch[_NBUF:_NBUF + _NST]
    sems = scratch[_NBUF + _NST:_NBUF + _NST + _NBUF]
    so = scratch[_NBUF + _NST + _NBUF:]
    xh = (x_hbm0, x_hbm1)

    starts_by_step = [[] for _ in range(len(_STEPS))]
    for q, t0 in _START_AT.items():
        starts_by_step[t0].append(q)

    def start_fetch(q):
        s = q % _NBUF
        pltpu.make_async_copy(
            xh[q % 2].at[_FETCHES[q]], bufs[s], sems[s]).start()

    def wait_fetch(q):
        s = q % _NBUF
        pltpu.make_async_copy(
            xh[q % 2].at[_FETCHES[q]], bufs[s], sems[s]).wait()

    for t, (orow, sq, pq) in enumerate(_STEPS):
        for q in starts_by_step[t]:
            start_fetch(q)
        for q in {sq, pq}:
            if _FIRST_USE[q] == t:
                wait_fetch(q)
        if t >= _NST:
            prow = _STEPS[t - _NST][0]
            pltpu.make_async_copy(
                st[t % _NST], y_hbm.at[prow], so[t % _NST]).wait()
        _row_mix(st[t % _NST], bufs[sq % _NBUF], bufs[pq % _NBUF], a_v, b_v)
        pltpu.make_async_copy(
            st[t % _NST], y_hbm.at[orow], so[t % _NST]).start()

    n = len(_STEPS)
    for t in range(n - _NST, n):
        pltpu.make_async_copy(
            st[t % _NST], y_hbm.at[_STEPS[t][0]], so[t % _NST]).wait()


def kernel(x):
    a = jnp.asarray(_A_NP)
    b = jnp.asarray(_B_NP)
    scratch = (
        [pltpu.VMEM((_H, _W, _C), jnp.float32)] * (_NBUF + _NST)
        + [pltpu.SemaphoreType.DMA] * (_NBUF + _NST)
    )
    y = pl.pallas_call(
        _mix_body,
        grid=(1,),
        in_specs=[
            pl.BlockSpec(memory_space=pltpu.MemorySpace.HBM),
            pl.BlockSpec(memory_space=pltpu.MemorySpace.HBM),
            pl.BlockSpec((_H, _W, _C), lambda i: (0, 0, 0)),
            pl.BlockSpec((_H, _W, _C), lambda i: (0, 0, 0)),
        ],
        out_specs=pl.BlockSpec(memory_space=pltpu.MemorySpace.HBM),
        out_shape=jax.ShapeDtypeStruct((_B, _H, _W, _C), jnp.float32),
        scratch_shapes=scratch,
    )(x, x, a, b)
    return y
